# baseline (speedup 1.0000x reference)
import functools

import jax
import jax.numpy as jnp
from jax import lax
from jax.experimental import pallas as pl
from jax.experimental.pallas import tpu as pltpu
from jax.experimental.pallas import tpu_sc as plsc

N = 10000
E = 320000
D = 128

NC = 2
NS = 16
NW = NC * NS
EPW = E // NW
K = 125
CH = EPW // K
GS = 16
NG = CH // GS
NB = 2
SLOT = 128
NP = 10240
RPS = NP // NS
ZR = 64
DEGW = 128

KA = 50
CHA = EPW // KA
GSA = 8
NGA = CHA // GSA
RB = 4
GD = 2
SLOTA = 56


def _sc_mesh():
    return plsc.VectorSubcoreMesh(core_axis_name="c", subcore_axis_name="s")


def _zero_stripe(zbuf, sh, s, width):
    def _fz(i, _):
        for j in range(width // 16):
            zbuf[i, pl.ds(j * 16, 16)] = jnp.zeros((16,), jnp.float32)
        return 0

    lax.fori_loop(0, ZR, _fz, 0)
    for t in range(RPS // ZR):
        pltpu.sync_copy(zbuf, sh.at[pl.ds(s * RPS + t * ZR, ZR)])


def _deg_body(dst_hbm, out_hbm, dstb, onesb, zbuf, deg_sh, sem):
    c = lax.axis_index("c")
    s = lax.axis_index("s")

    def _fill(i, _):
        for j in range(DEGW // 16):
            onesb[i, pl.ds(j * 16, 16)] = jnp.ones((16,), jnp.float32)
        return 0

    lax.fori_loop(0, K, _fill, 0)
    _zero_stripe(zbuf, deg_sh, s, DEGW)

    w = c * NS + s
    pltpu.sync_copy(dst_hbm.at[pl.ds(w * CH, CH)], dstb)
    plsc.subcore_barrier()

    def _chunk(i, _):
        pltpu.sync_copy(onesb, deg_sh.at[dstb.at[i]], add=True)
        return 0

    lax.fori_loop(0, CH, _chunk, 0)
    plsc.subcore_barrier()
    pltpu.sync_copy(deg_sh.at[pl.ds(s * RPS, RPS)],
                    out_hbm.at[pl.ds(c * NP + s * RPS, RPS)])


def _deg_call(dst2):
    kfn = pl.kernel(
        _deg_body,
        out_type=jax.ShapeDtypeStruct((NC * NP, DEGW), jnp.float32),
        mesh=_sc_mesh(),
        scratch_types=[
            pltpu.VMEM((CH, K), jnp.int32),
            pltpu.VMEM((K, DEGW), jnp.float32),
            pltpu.VMEM((ZR, DEGW), jnp.float32),
            pltpu.VMEM_SHARED((NP, DEGW), jnp.float32),
            pltpu.SemaphoreType.DMA,
        ],
        name="gcn_deg_sc",
    )
    return kfn(dst2)


def _agg_body(h_hbm, src_hbm, dst_hbm, out_hbm, sb0, sb1, db0, db1, rows,
              agg_sh, *sems):
    c = lax.axis_index("c")
    s = lax.axis_index("s")
    w = c * NS + s

    sbufs = (sb0, sb1)
    dbufs = (db0, db1)
    ssems = sems[0:2]
    dsems = sems[2:4]
    gsem = sems[4:4 + RB]
    ssem = sems[4 + RB:4 + 2 * RB]

    def _fz(i, _):
        for j in range(D // 16):
            rows[i, pl.ds(j * 16, 16)] = jnp.zeros((16,), jnp.float32)
        return 0

    lax.fori_loop(0, 64, _fz, 0)
    for t in range(RPS // 64):
        pltpu.sync_copy(rows.at[pl.ds(0, 64)],
                        agg_sh.at[pl.ds(s * RPS + t * 64, 64)])

    pltpu.sync_copy(src_hbm.at[pl.ds(w * CHA, GSA)], sb0)
    pltpu.async_copy(src_hbm.at[pl.ds(w * CHA + GSA, GSA)], sb1, ssems[1])
    pltpu.sync_copy(dst_hbm.at[pl.ds(w * CHA, GSA)], db0)
    pltpu.async_copy(dst_hbm.at[pl.ds(w * CHA + GSA, GSA)], db1, dsems[1])
    plsc.subcore_barrier()

    def _slot(b):
        return rows.at[pl.ds(b * SLOTA, KA)]

    def _gather(j, b):
        gref = h_hbm.at[sbufs[(j // GSA) % 2].at[j % GSA]]
        pltpu.async_copy(gref, _slot(b), gsem[b])
        return gref

    pending = [None] * RB
    for j in range(GD):
        pending[j] = _gather(j, j)

    scat = [None] * RB
    for i in range(CHA):
        b = i % RB
        blkd, td = i // GSA, i % GSA
        if td == 0 and blkd >= 1:
            pltpu.make_async_copy(
                dst_hbm.at[pl.ds(w * CHA + blkd * GSA, GSA)],
                dbufs[blkd % 2], dsems[blkd % 2]).wait()
        pltpu.make_async_copy(pending[b], _slot(b), gsem[b]).wait()
        dref = agg_sh.at[dbufs[blkd % 2].at[td]]
        pltpu.async_copy(_slot(b), dref, ssem[b], add=True)
        scat[b] = dref
        if td == 2 and 1 <= blkd < NGA - 1:
            pltpu.async_copy(
                dst_hbm.at[pl.ds(w * CHA + (blkd + 1) * GSA, GSA)],
                dbufs[(blkd + 1) % 2], dsems[(blkd + 1) % 2])
        j = i + GD
        if j < CHA:
            bj = j % RB
            if j >= RB:
                pltpu.make_async_copy(_slot(bj), scat[bj], ssem[bj]).wait()
            blk, t = j // GSA, j % GSA
            if t == 0 and blk >= 1:
                pltpu.make_async_copy(
                    src_hbm.at[pl.ds(w * CHA + blk * GSA, GSA)],
                    sbufs[blk % 2], ssems[blk % 2]).wait()
            if t == GD and 1 <= blk < NGA - 1:
                pltpu.async_copy(
                    src_hbm.at[pl.ds(w * CHA + (blk + 1) * GSA, GSA)],
                    sbufs[(blk + 1) % 2], ssems[(blk + 1) % 2])
            pending[bj] = _gather(j, bj)

    for i in range(CHA - RB, CHA):
        b = i % RB
        pltpu.make_async_copy(_slot(b), scat[b], ssem[b]).wait()
    plsc.subcore_barrier()
    pltpu.sync_copy(agg_sh.at[pl.ds(s * RPS, RPS)],
                    out_hbm.at[pl.ds(c * NP + s * RPS, RPS)])


def _agg_call(h, src2, dst2):
    kfn = pl.kernel(
        _agg_body,
        out_type=jax.ShapeDtypeStruct((NC * NP, D), jnp.float32),
        mesh=_sc_mesh(),
        scratch_types=[
            pltpu.VMEM((GSA, KA), jnp.int32),
            pltpu.VMEM((GSA, KA), jnp.int32),
            pltpu.VMEM((GSA, KA), jnp.int32),
            pltpu.VMEM((GSA, KA), jnp.int32),
            pltpu.VMEM((RB * SLOTA, D), jnp.float32),
            pltpu.VMEM_SHARED((NP, D), jnp.float32),
        ] + [pltpu.SemaphoreType.DMA] * (4 + 2 * RB),
        name="gcn_agg_sc",
    )
    return kfn(h, src2, dst2)


BM = 1000


def _mm_body(x_ref, w_ref, o_ref):
    o_ref[...] = jnp.dot(x_ref[...], w_ref[...],
                         preferred_element_type=jnp.float32)


def _matmul(x, W):
    return pl.pallas_call(
        _mm_body,
        grid=(N // BM,),
        in_specs=[pl.BlockSpec((BM, D), lambda i: (i, 0)),
                  pl.BlockSpec((D, D), lambda i: (0, 0))],
        out_specs=pl.BlockSpec((BM, D), lambda i: (i, 0)),
        out_shape=jax.ShapeDtypeStruct((N, D), jnp.float32),
    )(x, W)


def _deg_inv(da, db):
    deg = (da + db)[:, 0:1]
    return 1.0 / jnp.maximum(deg, 1.0)


def _fused_body(pa_ref, pb_ref, da_ref, db_ref, w_ref, o_ref):
    t = (pa_ref[...] + pb_ref[...]) * _deg_inv(da_ref[...], db_ref[...])
    t = jnp.maximum(t, 0.0)
    mu = jnp.mean(t, axis=-1, keepdims=True)
    var = jnp.mean((t - mu) ** 2, axis=-1, keepdims=True)
    t = (t - mu) * lax.rsqrt(var + 1e-9)
    o_ref[...] = jnp.dot(t, w_ref[...], preferred_element_type=jnp.float32)


def _fused(pa, pb, da, db, W):
    return pl.pallas_call(
        _fused_body,
        grid=(N // BM,),
        in_specs=[pl.BlockSpec((BM, D), lambda i: (i, 0)),
                  pl.BlockSpec((BM, D), lambda i: (i, 0)),
                  pl.BlockSpec((BM, DEGW), lambda i: (i, 0)),
                  pl.BlockSpec((BM, DEGW), lambda i: (i, 0)),
                  pl.BlockSpec((D, D), lambda i: (0, 0))],
        out_specs=pl.BlockSpec((BM, D), lambda i: (i, 0)),
        out_shape=jax.ShapeDtypeStruct((N, D), jnp.float32),
    )(pa, pb, da, db, W)


def _final_body(pa_ref, pb_ref, da_ref, db_ref, o_ref):
    o_ref[...] = (pa_ref[...] + pb_ref[...]) * _deg_inv(da_ref[...],
                                                        db_ref[...])


def _final(pa, pb, da, db):
    return pl.pallas_call(
        _final_body,
        grid=(N // BM,),
        in_specs=[pl.BlockSpec((BM, D), lambda i: (i, 0)),
                  pl.BlockSpec((BM, D), lambda i: (i, 0)),
                  pl.BlockSpec((BM, DEGW), lambda i: (i, 0)),
                  pl.BlockSpec((BM, DEGW), lambda i: (i, 0))],
        out_specs=pl.BlockSpec((BM, D), lambda i: (i, 0)),
        out_shape=jax.ShapeDtypeStruct((N, D), jnp.float32),
    )(pa, pb, da, db)


def kernel(sparse_adj, feats, W0, W1, W2):
    dst2 = sparse_adj[1].reshape(E // K, K)
    src2a = sparse_adj[0].reshape(E // KA, KA)
    dst2a = sparse_adj[1].reshape(E // KA, KA)

    degp = _deg_call(dst2)
    da, db = degp[:N], degp[NP:NP + N]

    h = _matmul(feats, W0)
    p = _agg_call(h, src2a, dst2a)
    h = _fused(p[:N], p[NP:NP + N], da, db, W1)
    p = _agg_call(h, src2a, dst2a)
    h = _fused(p[:N], p[NP:NP + N], da, db, W2)
    p = _agg_call(h, src2a, dst2a)
    return _final(p[:N], p[NP:NP + N], da, db)
